# Initial kernel scaffold; baseline (speedup 1.0000x reference)
import jax
import jax.numpy as jnp
from jax.experimental import pallas as pl


def kernel(x_movie, x_user, edge_rates, edge_rev, tuples_coo, user_emb, W1l, b1l, W1r, W2l, b2l, W2r, W3l, b3l, W3r, Wlin1, blin1, Wlin2, blin2):
    raise NotImplementedError("write your pallas kernel here")


# speedup vs baseline: 5.7939x; 5.7939x over previous
import functools

import jax
import jax.numpy as jnp
from jax import lax
from jax.experimental import pallas as pl
from jax.experimental.pallas import tpu as pltpu
from jax.experimental.pallas import tpu_sc as plsc

N = 10000
D = 128
DH = D // 2
E = 500000
NC, NS, L = 2, 16, 16
CHUNK = 128
NCH = -(-E // (NS * CHUNK))
EPAD = NS * NCH * CHUNK
NROWS = 10240
STRIPE = NROWS // NS

_f32 = jnp.float32
_i32 = jnp.int32


def _segsum_body(src_h, dst_h, tbl_h, sum_h, cnt_h,
                 src_v, dst_v, rows_v, ones_v, zrow_v, acc_sh, cnt_sh, sem):
    cid = lax.axis_index("c")
    sid = lax.axis_index("s")

    zero16 = jnp.zeros((L,), _f32)
    ones16 = jnp.ones((L,), _f32)

    @pl.loop(0, CHUNK)
    def _(i):
        for j in range(DH // L):
            rows_v[i, pl.ds(j * L, L)] = zero16
        ones_v[i, :] = ones16
        zrow_v[i, :] = zero16

    for t in range(STRIPE // CHUNK):
        sl = pl.ds(sid * STRIPE + t * CHUNK, CHUNK)
        pltpu.sync_copy(rows_v, acc_sh.at[sl])
        pltpu.sync_copy(zrow_v, cnt_sh.at[sl])
    plsc.subcore_barrier()

    pltpu.sync_copy(src_h.at[sid], src_v)
    pltpu.sync_copy(dst_h.at[sid], dst_v)

    @pl.loop(0, NCH)
    def _(j):
        pltpu.async_copy(tbl_h.at[cid].at[src_v.at[j]], rows_v, sem).wait()
        pltpu.sync_copy(rows_v, acc_sh.at[dst_v.at[j]], add=True)

        @pl.when(cid == 0)
        def _():
            pltpu.sync_copy(ones_v, cnt_sh.at[dst_v.at[j]], add=True)

    plsc.subcore_barrier()

    for t in range(STRIPE // CHUNK):
        sl = pl.ds(sid * STRIPE + t * CHUNK, CHUNK)
        pltpu.sync_copy(acc_sh.at[sl], sum_h.at[cid, sl])

        @pl.when(cid == 0)
        def _():
            pltpu.sync_copy(cnt_sh.at[sl], cnt_h.at[sl])


@functools.cache
def _get_segsum():
    return pl.kernel(
        _segsum_body,
        out_type=[
            jax.ShapeDtypeStruct((NC, NROWS, DH), _f32),
            jax.ShapeDtypeStruct((NROWS, L), _f32),
        ],
        mesh=plsc.VectorSubcoreMesh(core_axis_name="c", subcore_axis_name="s",
                                    num_cores=NC, num_subcores=NS),
        compiler_params=pltpu.CompilerParams(use_tc_tiling_on_sc=False),
        scratch_types=[
            pltpu.VMEM((NCH, CHUNK), _i32),
            pltpu.VMEM((NCH, CHUNK), _i32),
            pltpu.VMEM((CHUNK, DH), _f32),
            pltpu.VMEM((CHUNK, L), _f32),
            pltpu.VMEM((CHUNK, L), _f32),
            pltpu.VMEM_SHARED((NROWS, DH), _f32),
            pltpu.VMEM_SHARED((NROWS, L), _f32),
            pltpu.SemaphoreType.DMA,
        ],
    )


_BM = 1000
_GRID = N // _BM


def _dotT(a, b):
    return lax.dot_general(a, b, (((1,), (1,)), ((), ())),
                           preferred_element_type=_f32)


def _full(shape):
    return pl.BlockSpec(shape, lambda i: (0,) * len(shape))


def _rows(shape):
    if len(shape) == 3:
        return pl.BlockSpec(shape, lambda i: (0, i, 0))
    return pl.BlockSpec(shape, lambda i: (i, 0))


def _split(p_ref, x):
    p_ref[0] = x[:, :DH]
    p_ref[1] = x[:, DH:]


def _mm0_body(x_ref, w_ref, o_ref):
    _split(o_ref, _dotT(x_ref[...], w_ref[...]))


_mm0 = pl.pallas_call(
    _mm0_body,
    grid=(_GRID,),
    in_specs=[_rows((_BM, D)), _full((D, D))],
    out_specs=_rows((NC, _BM, DH)),
    out_shape=jax.ShapeDtypeStruct((NC, N, DH), _f32),
)


def _st1_body(s_ref, c_ref, ue_ref, w1r_ref, b1l_ref, w2l_ref, ux_ref, p2_ref):
    s = jnp.concatenate([s_ref[0], s_ref[1]], axis=1)
    inv = 1.0 / jnp.maximum(c_ref[:, 0:1], 1.0)
    r1 = _dotT(ue_ref[...], w1r_ref[...])
    ux = jnp.maximum(s * inv + b1l_ref[...] + r1, 0.0)
    ux_ref[...] = ux
    _split(p2_ref, _dotT(ux, w2l_ref[...]))


_st1 = pl.pallas_call(
    _st1_body,
    grid=(_GRID,),
    in_specs=[_rows((NC, _BM, DH)), _rows((_BM, L)), _full((1, D)),
              _full((D, D)), _full((1, D)), _full((D, D))],
    out_specs=[_rows((_BM, D)), _rows((NC, _BM, DH))],
    out_shape=[jax.ShapeDtypeStruct((N, D), _f32),
               jax.ShapeDtypeStruct((NC, N, DH), _f32)],
)


def _st2_body(s_ref, c_ref, xm_ref, w2r_ref, b2l_ref, w3l_ref, wl2_ref,
              bl2_ref, p3_ref, zm_ref):
    s = jnp.concatenate([s_ref[0], s_ref[1]], axis=1)
    inv = 1.0 / jnp.maximum(c_ref[:, 0:1], 1.0)
    mx = jnp.maximum(s * inv + b2l_ref[...] + _dotT(xm_ref[...], w2r_ref[...]),
                     0.0)
    _split(p3_ref, _dotT(mx, w3l_ref[...]))
    zm_ref[...] = _dotT(mx, wl2_ref[...]) + bl2_ref[...]


_st2 = pl.pallas_call(
    _st2_body,
    grid=(_GRID,),
    in_specs=[_rows((NC, _BM, DH)), _rows((_BM, L)), _rows((_BM, D)),
              _full((D, D)), _full((1, D)), _full((D, D)),
              _full((64, D)), _full((1, 64))],
    out_specs=[_rows((NC, _BM, DH)), _rows((_BM, 64))],
    out_shape=[jax.ShapeDtypeStruct((NC, N, DH), _f32),
               jax.ShapeDtypeStruct((N, 64), _f32)],
)


def _st3_body(s_ref, c_ref, ux_ref, w3r_ref, b3l_ref, wl1_ref, bl1_ref,
              zu_ref):
    s = jnp.concatenate([s_ref[0], s_ref[1]], axis=1)
    inv = 1.0 / jnp.maximum(c_ref[:, 0:1], 1.0)
    ux2 = jnp.maximum(s * inv + b3l_ref[...] + _dotT(ux_ref[...], w3r_ref[...]),
                      0.0)
    zu_ref[...] = _dotT(ux2, wl1_ref[...]) + bl1_ref[...]


_st3 = pl.pallas_call(
    _st3_body,
    grid=(_GRID,),
    in_specs=[_rows((NC, _BM, DH)), _rows((_BM, L)), _rows((_BM, D)),
              _full((D, D)), _full((1, D)), _full((64, D)), _full((1, 64))],
    out_specs=_rows((_BM, 64)),
    out_shape=jax.ShapeDtypeStruct((N, 64), _f32),
)


def _pad_edges(src, dst):
    pad = EPAD - E
    src3 = jnp.concatenate([src.astype(_i32), jnp.zeros((pad,), _i32)])
    dst3 = jnp.concatenate([dst.astype(_i32), jnp.full((pad,), N, _i32)])
    return src3.reshape(NS, NCH, CHUNK), dst3.reshape(NS, NCH, CHUNK)


def kernel(x_movie, x_user, edge_rates, edge_rev, tuples_coo, user_emb,
           W1l, b1l, W1r, W2l, b2l, W2r, W3l, b3l, W3r,
           Wlin1, blin1, Wlin2, blin2):
    n_users = x_user.shape[0]
    srcR, dstR = _pad_edges(edge_rev[0], edge_rev[1])
    srcA, dstA = _pad_edges(edge_rates[0], edge_rates[1])

    ue = user_emb.reshape(1, D)
    b1 = b1l.reshape(1, D)
    b2 = b2l.reshape(1, D)
    b3 = b3l.reshape(1, D)
    bz1 = blin1.reshape(1, 64)
    bz2 = blin2.reshape(1, 64)

    p1 = _mm0(x_movie, W1l)
    s1, c1 = _get_segsum()(srcR, dstR, p1)
    ux, p2 = _st1(s1, c1, ue, W1r, b1, W2l)

    s2, c2 = _get_segsum()(srcA, dstA, p2)
    p3, zm = _st2(s2, c2, x_movie, W2r, b2, W3l, Wlin2, bz2)

    s3, _ = _get_segsum()(srcR, dstR, p3)
    zu = _st3(s3, c1, ux, W3r, b3, Wlin1, bz1)

    X = jnp.concatenate([zu, zm], axis=0)
    new_index = jnp.vstack((tuples_coo[0], tuples_coo[1] + n_users))
    return (X, new_index)
